# baseline (speedup 1.0000x reference)
import functools
import jax
import jax.numpy as jnp
from jax import lax
from jax.experimental import pallas as pl
from jax.experimental.pallas import tpu as pltpu
from jax.experimental.pallas import tpu_sc as plsc

B_TOK = 4096
SEQ = 200
EMB = 64
VOCAB_ROWS = 1000000
SCALE = 8.0

NC = 2
NS = 16
NW = NC * NS
BLK = B_TOK // NW
KB = EMB // 8
PITCH = BLK + 1
NBUF = 4
TCHUNK = 2048
HALF = VOCAB_ROWS // 2

_mesh = plsc.VectorSubcoreMesh(core_axis_name="c", subcore_axis_name="s")


@functools.partial(
    pl.kernel,
    mesh=_mesh,
    out_type=jax.ShapeDtypeStruct((SEQ, KB, NW, 8, BLK), jnp.float32),
    scratch_types=[
        pltpu.VMEM((SEQ, BLK), jnp.int32),
        pltpu.VMEM((NBUF, BLK, EMB), jnp.float32),
        pltpu.VMEM((NBUF, EMB, PITCH), jnp.float32),
    ]
    + [pltpu.SemaphoreType.DMA] * (2 * NBUF),
    compiler_params=pltpu.CompilerParams(
        use_tc_tiling_on_sc=False, needs_layout_passes=False),
)
def _emb_lookup(tok_hbm, table_hbm, out_hbm, idx_v, gbuf, sbuf, *sems):
    gsem = sems[:NBUF]
    ssem = sems[NBUF:]
    wid = lax.axis_index("s") * NC + lax.axis_index("c")
    pltpu.sync_copy(tok_hbm.at[:, wid], idx_v)
    iota16 = lax.iota(jnp.int32, 16)
    rows_q = [iota16 + q * 16 for q in range(EMB // 16)]

    def start_gather(s, b):
        pltpu.async_copy(table_hbm.at[idx_v.at[s]], gbuf.at[b], gsem[b])

    def wait_gather(s, b):
        pltpu.make_async_copy(table_hbm.at[idx_v.at[s]], gbuf.at[b],
                              gsem[b]).wait()

    def start_store(s, b):
        for kb in range(KB):
            pltpu.async_copy(sbuf.at[b, pl.ds(kb * 8, 8), pl.ds(0, BLK)],
                             out_hbm.at[s, kb, wid], ssem[b])

    def wait_store(s, b):
        for kb in range(KB):
            pltpu.make_async_copy(sbuf.at[b, pl.ds(kb * 8, 8), pl.ds(0, BLK)],
                                  out_hbm.at[s, kb, wid], ssem[b]).wait()

    def xform(b):
        @plsc.parallel_loop(0, BLK, 1, unroll=8)
        def tok_body(tb):
            cols = iota16 * 0 + tb
            for q in range(EMB // 16):
                v = gbuf[b, tb, pl.ds(q * 16, 16)]
                plsc.store_scatter(sbuf.at[b], [rows_q[q], cols], v)

    for b in range(NBUF):
        start_gather(b, b)
    for b in range(NBUF):
        wait_gather(b, b)
        xform(b)
        start_gather(b + NBUF, b)
        start_store(b, b)

    def body(t, carry):
        for b in range(NBUF):
            s = t * NBUF + b
            wait_gather(s, b)
            wait_store(s, b)
            xform(b)

            @pl.when(s + NBUF < SEQ)
            def _():
                start_gather(s + NBUF, b)

            start_store(s, b)
        return carry

    lax.fori_loop(1, SEQ // NBUF, body, 0)

    for b in range(NBUF):
        wait_store(SEQ - NBUF + b, b)


def _tc_scale_transpose(tt_ref, out_ref):
    w = jnp.concatenate([tt_ref[:, 0, :], tt_ref[:, 1, :]], axis=0)
    out_ref[...] = jnp.transpose(w) * SCALE


_scale_transpose = pl.pallas_call(
    _tc_scale_transpose,
    grid=(pl.cdiv(HALF, TCHUNK),),
    in_specs=[pl.BlockSpec((EMB, 2, TCHUNK), lambda i: (0, 0, i))],
    out_specs=pl.BlockSpec((TCHUNK, 2 * EMB), lambda i: (i, 0)),
    out_shape=jax.ShapeDtypeStruct((HALF, 2 * EMB), jnp.float32),
)


def kernel(tokens, table):
    tok = tokens.astype(jnp.int32)
    tok = jnp.where(tok < HALF, 2 * tok, 2 * tok - (VOCAB_ROWS - 1))
    tok_t = tok.T.reshape(SEQ, NW, BLK)
    tt = table.T.reshape(EMB, 2, HALF)
    scaled = _scale_transpose(tt).reshape(VOCAB_ROWS, EMB)
    flat = _emb_lookup(tok_t, scaled)
    return flat.transpose(2, 4, 0, 1, 3).reshape(B_TOK, SEQ, EMB)


# speedup vs baseline: 3.3372x; 3.3372x over previous
import functools
import jax
import jax.numpy as jnp
from jax import lax
from jax.experimental import pallas as pl
from jax.experimental.pallas import tpu as pltpu
from jax.experimental.pallas import tpu_sc as plsc

B_TOK = 4096
SEQ = 200
EMB = 64
VOCAB_ROWS = 1000000
SCALE = 8.0

NC = 2
NS = 16
NW = NC * NS
BLK = B_TOK // NW
KB = EMB // 8
PITCH = BLK + 1
NBUF = 4
TCHUNK = 2048
NTBLK = 245
PHALF = NTBLK * TCHUNK
STORED_ROWS = 2 * PHALF

_mesh = plsc.VectorSubcoreMesh(core_axis_name="c", subcore_axis_name="s")


@functools.partial(
    pl.kernel,
    mesh=_mesh,
    out_type=jax.ShapeDtypeStruct((SEQ, KB, NW, 8, BLK), jnp.float32),
    scratch_types=[
        pltpu.VMEM((SEQ, BLK), jnp.int32),
        pltpu.VMEM((NBUF, BLK, EMB), jnp.float32),
        pltpu.VMEM((NBUF, EMB, PITCH), jnp.float32),
    ]
    + [pltpu.SemaphoreType.DMA] * (2 * NBUF),
    compiler_params=pltpu.CompilerParams(
        use_tc_tiling_on_sc=False, needs_layout_passes=False),
)
def _emb_lookup(tok_hbm, table_hbm, out_hbm, idx_v, gbuf, sbuf, *sems):
    gsem = sems[:NBUF]
    ssem = sems[NBUF:]
    wid = lax.axis_index("s") * NC + lax.axis_index("c")
    pltpu.sync_copy(tok_hbm.at[:, wid], idx_v)
    iota16 = lax.iota(jnp.int32, 16)
    rows_q = [iota16 + q * 16 for q in range(EMB // 16)]

    def start_gather(s, b):
        pltpu.async_copy(table_hbm.at[idx_v.at[s]], gbuf.at[b], gsem[b])

    def wait_gather(s, b):
        pltpu.make_async_copy(table_hbm.at[idx_v.at[s]], gbuf.at[b],
                              gsem[b]).wait()

    def start_store(s, b):
        for kb in range(KB):
            pltpu.async_copy(sbuf.at[b, pl.ds(kb * 8, 8), pl.ds(0, BLK)],
                             out_hbm.at[s, kb, wid], ssem[b])

    def wait_store(s, b):
        for kb in range(KB):
            pltpu.make_async_copy(sbuf.at[b, pl.ds(kb * 8, 8), pl.ds(0, BLK)],
                                  out_hbm.at[s, kb, wid], ssem[b]).wait()

    def xform(b):
        @plsc.parallel_loop(0, BLK, 1, unroll=8)
        def tok_body(tb):
            cols = iota16 * 0 + tb
            for q in range(EMB // 16):
                v = gbuf[b, tb, pl.ds(q * 16, 16)]
                plsc.store_scatter(sbuf.at[b], [rows_q[q], cols], v)

    for b in range(NBUF):
        start_gather(b, b)
    for b in range(NBUF):
        wait_gather(b, b)
        xform(b)
        start_gather(b + NBUF, b)
        start_store(b, b)

    def body(t, carry):
        for b in range(NBUF):
            s = t * NBUF + b
            wait_gather(s, b)
            wait_store(s, b)
            xform(b)

            @pl.when(s + NBUF < SEQ)
            def _():
                start_gather(s + NBUF, b)

            start_store(s, b)
        return carry

    lax.fori_loop(1, SEQ // NBUF, body, 0)

    for b in range(NBUF):
        wait_store(SEQ - NBUF + b, b)


def _tc_scale_transpose(lo_ref, hi_ref, out_ref):
    w = jnp.concatenate([lo_ref[...], hi_ref[...]], axis=0)
    out_ref[...] = jnp.transpose(w) * SCALE


_scale_transpose = pl.pallas_call(
    _tc_scale_transpose,
    grid=(NTBLK,),
    in_specs=[
        pl.BlockSpec((EMB, TCHUNK), lambda i: (0, i)),
        pl.BlockSpec((EMB, TCHUNK),
                     lambda i: (0, jnp.minimum(i + NTBLK, NTBLK * 2 - 2))),
    ],
    out_specs=pl.BlockSpec((TCHUNK, 2 * EMB), lambda i: (i, 0)),
    out_shape=jax.ShapeDtypeStruct((PHALF, 2 * EMB), jnp.float32),
)


def kernel(tokens, table):
    tok = tokens.astype(jnp.int32)
    tok = jnp.where(tok < PHALF, 2 * tok, 2 * tok - (STORED_ROWS - 1))
    tok_t = tok.T.reshape(SEQ, NW, BLK)
    tt = table.T
    scaled = _scale_transpose(tt, tt).reshape(STORED_ROWS, EMB)
    flat = _emb_lookup(tok_t, scaled)
    return flat.transpose(2, 4, 0, 1, 3).reshape(B_TOK, SEQ, EMB)


# speedup vs baseline: 4.0215x; 1.2050x over previous
import functools
import jax
import jax.numpy as jnp
from jax import lax
from jax.experimental import pallas as pl
from jax.experimental.pallas import tpu as pltpu
from jax.experimental.pallas import tpu_sc as plsc

B_TOK = 4096
SEQ = 200
EMB = 64
VOCAB_ROWS = 1000000
SCALE = 8.0

NC = 2
NS = 16
NW = NC * NS
BLK = B_TOK // NW
KB = EMB // 8
PITCH = BLK + 1
NBUF = 4
TCHUNK = 4096
NTBLK = 123
PHALF = NTBLK * TCHUNK
STORED_ROWS = 2 * PHALF

_mesh = plsc.VectorSubcoreMesh(core_axis_name="c", subcore_axis_name="s")


@functools.partial(
    pl.kernel,
    mesh=_mesh,
    out_type=jax.ShapeDtypeStruct((SEQ, KB, NW, 8, BLK), jnp.float32),
    scratch_types=[
        pltpu.VMEM((SEQ, BLK), jnp.int32),
        pltpu.VMEM((NBUF, BLK, EMB), jnp.float32),
        pltpu.VMEM((NBUF, EMB, PITCH), jnp.float32),
    ]
    + [pltpu.SemaphoreType.DMA] * (2 * NBUF),
    compiler_params=pltpu.CompilerParams(
        use_tc_tiling_on_sc=False, needs_layout_passes=False),
)
def _emb_lookup(tok_hbm, table_hbm, out_hbm, idx_v, gbuf, sbuf, *sems):
    gsem = sems[:NBUF]
    ssem = sems[NBUF:]
    wid = lax.axis_index("s") * NC + lax.axis_index("c")
    pltpu.sync_copy(tok_hbm.at[:, wid], idx_v)
    iota16 = lax.iota(jnp.int32, 16)
    rows_q = [iota16 + q * 16 for q in range(EMB // 16)]

    def start_gather(s, b):
        pltpu.async_copy(table_hbm.at[idx_v.at[s]], gbuf.at[b], gsem[b])

    def wait_gather(s, b):
        pltpu.make_async_copy(table_hbm.at[idx_v.at[s]], gbuf.at[b],
                              gsem[b]).wait()

    def start_store(s, b):
        for kb in range(KB):
            pltpu.async_copy(sbuf.at[b, pl.ds(kb * 8, 8), pl.ds(0, BLK)],
                             out_hbm.at[s, kb, wid], ssem[b])

    def wait_store(s, b):
        for kb in range(KB):
            pltpu.make_async_copy(sbuf.at[b, pl.ds(kb * 8, 8), pl.ds(0, BLK)],
                                  out_hbm.at[s, kb, wid], ssem[b]).wait()

    def xform(b):
        @plsc.parallel_loop(0, BLK, 1, unroll=8)
        def tok_body(tb):
            cols = iota16 * 0 + tb
            for q in range(EMB // 16):
                v = gbuf[b, tb, pl.ds(q * 16, 16)]
                plsc.store_scatter(sbuf.at[b], [rows_q[q], cols], v)

    for b in range(NBUF):
        start_gather(b, b)
    for b in range(NBUF):
        wait_gather(b, b)
        xform(b)
        start_gather(b + NBUF, b)
        start_store(b, b)

    def body(t, carry):
        for b in range(NBUF):
            s = t * NBUF + b
            wait_gather(s, b)
            wait_store(s, b)
            xform(b)

            @pl.when(s + NBUF < SEQ)
            def _():
                start_gather(s + NBUF, b)

            start_store(s, b)
        return carry

    lax.fori_loop(1, SEQ // NBUF, body, 0)

    for b in range(NBUF):
        wait_store(SEQ - NBUF + b, b)


def _tc_scale_transpose(lo_ref, hi_ref, out_ref):
    w = jnp.concatenate([lo_ref[...], hi_ref[...]], axis=0)
    out_ref[...] = jnp.transpose(w) * SCALE


_scale_transpose = pl.pallas_call(
    _tc_scale_transpose,
    grid=(NTBLK,),
    in_specs=[
        pl.BlockSpec((EMB, TCHUNK), lambda i: (0, i)),
        pl.BlockSpec((EMB, TCHUNK),
                     lambda i: (0, jnp.minimum(i + NTBLK, NTBLK * 2 - 2))),
    ],
    out_specs=pl.BlockSpec((TCHUNK, 2 * EMB), lambda i: (i, 0)),
    out_shape=jax.ShapeDtypeStruct((PHALF, 2 * EMB), jnp.float32),
)


def kernel(tokens, table):
    tok = tokens.astype(jnp.int32)
    tok = jnp.where(tok < PHALF, 2 * tok, 2 * tok - (STORED_ROWS - 1))
    tok_t = tok.T.reshape(SEQ, NW, BLK)
    tt = table.T
    scaled = _scale_transpose(tt, tt).reshape(STORED_ROWS, EMB)
    flat = _emb_lookup(tok_t, scaled)
    return flat.transpose(2, 4, 0, 1, 3).reshape(B_TOK, SEQ, EMB)


# speedup vs baseline: 4.3463x; 1.0808x over previous
import functools
import jax
import jax.numpy as jnp
from jax import lax
from jax.experimental import pallas as pl
from jax.experimental.pallas import tpu as pltpu
from jax.experimental.pallas import tpu_sc as plsc

B_TOK = 4096
SEQ = 200
EMB = 64
VOCAB_ROWS = 1000000
SCALE = 8.0

NC = 2
NS = 16
NW = NC * NS
BLK = B_TOK // NW
KB = EMB // 8
PITCH = BLK + 1
NBUF = 4
TCHUNK = 8192
NTBLK = 62
PHALF = NTBLK * TCHUNK
STORED_ROWS = 2 * PHALF

_mesh = plsc.VectorSubcoreMesh(core_axis_name="c", subcore_axis_name="s")


@functools.partial(
    pl.kernel,
    mesh=_mesh,
    out_type=jax.ShapeDtypeStruct((SEQ, KB, NW, 8, BLK), jnp.float32),
    scratch_types=[
        pltpu.VMEM((SEQ, BLK), jnp.int32),
        pltpu.VMEM((NBUF, BLK, EMB), jnp.float32),
        pltpu.VMEM((NBUF, EMB, PITCH), jnp.float32),
    ]
    + [pltpu.SemaphoreType.DMA] * (2 * NBUF),
    compiler_params=pltpu.CompilerParams(
        use_tc_tiling_on_sc=False, needs_layout_passes=False),
)
def _emb_lookup(tok_hbm, table_hbm, out_hbm, idx_v, gbuf, sbuf, *sems):
    gsem = sems[:NBUF]
    ssem = sems[NBUF:]
    wid = lax.axis_index("s") * NC + lax.axis_index("c")
    pltpu.sync_copy(tok_hbm.at[:, wid], idx_v)
    iota16 = lax.iota(jnp.int32, 16)
    rows_q = [iota16 + q * 16 for q in range(EMB // 16)]

    def start_gather(s, b):
        pltpu.async_copy(table_hbm.at[idx_v.at[s]], gbuf.at[b], gsem[b])

    def wait_gather(s, b):
        pltpu.make_async_copy(table_hbm.at[idx_v.at[s]], gbuf.at[b],
                              gsem[b]).wait()

    def start_store(s, b):
        for kb in range(KB):
            pltpu.async_copy(sbuf.at[b, pl.ds(kb * 8, 8), pl.ds(0, BLK)],
                             out_hbm.at[s, kb, wid], ssem[b])

    def wait_store(s, b):
        for kb in range(KB):
            pltpu.make_async_copy(sbuf.at[b, pl.ds(kb * 8, 8), pl.ds(0, BLK)],
                                  out_hbm.at[s, kb, wid], ssem[b]).wait()

    def xform(b):
        @plsc.parallel_loop(0, BLK, 1, unroll=8)
        def tok_body(tb):
            cols = iota16 * 0 + tb
            for q in range(EMB // 16):
                v = gbuf[b, tb, pl.ds(q * 16, 16)]
                plsc.store_scatter(sbuf.at[b], [rows_q[q], cols], v)

    for b in range(NBUF):
        start_gather(b, b)
    for b in range(NBUF):
        wait_gather(b, b)
        xform(b)
        start_gather(b + NBUF, b)
        start_store(b, b)

    def body(t, carry):
        for b in range(NBUF):
            s = t * NBUF + b
            wait_gather(s, b)
            wait_store(s, b)
            xform(b)

            @pl.when(s + NBUF < SEQ)
            def _():
                start_gather(s + NBUF, b)

            start_store(s, b)
        return carry

    lax.fori_loop(1, SEQ // NBUF, body, 0)

    for b in range(NBUF):
        wait_store(SEQ - NBUF + b, b)


def _tc_scale_transpose(lo_ref, hi_ref, out_ref):
    w = jnp.concatenate([lo_ref[...], hi_ref[...]], axis=0)
    out_ref[...] = jnp.transpose(w) * SCALE


_scale_transpose = pl.pallas_call(
    _tc_scale_transpose,
    grid=(NTBLK,),
    in_specs=[
        pl.BlockSpec((EMB, TCHUNK), lambda i: (0, i)),
        pl.BlockSpec((EMB, TCHUNK),
                     lambda i: (0, jnp.minimum(i + NTBLK, NTBLK * 2 - 2))),
    ],
    out_specs=pl.BlockSpec((TCHUNK, 2 * EMB), lambda i: (i, 0)),
    out_shape=jax.ShapeDtypeStruct((PHALF, 2 * EMB), jnp.float32),
)


def kernel(tokens, table):
    tok = tokens.astype(jnp.int32)
    tok = jnp.where(tok < PHALF, 2 * tok, 2 * tok - (STORED_ROWS - 1))
    tok_t = tok.T.reshape(SEQ, NW, BLK)
    tt = table.T
    scaled = _scale_transpose(tt, tt).reshape(STORED_ROWS, EMB)
    flat = _emb_lookup(tok_t, scaled)
    return flat.transpose(2, 4, 0, 1, 3).reshape(B_TOK, SEQ, EMB)


# speedup vs baseline: 4.4368x; 1.0208x over previous
import functools
import jax
import jax.numpy as jnp
from jax import lax
from jax.experimental import pallas as pl
from jax.experimental.pallas import tpu as pltpu
from jax.experimental.pallas import tpu_sc as plsc

B_TOK = 4096
SEQ = 200
EMB = 64
VOCAB_ROWS = 1000000
SCALE = 8.0

NC = 2
NS = 16
NW = NC * NS
BLK = B_TOK // NW
KB = EMB // 8
PITCH = BLK + 1
NBUF = 4
TCHUNK = 16384
NTBLK = 31
PHALF = NTBLK * TCHUNK
STORED_ROWS = 2 * PHALF

_mesh = plsc.VectorSubcoreMesh(core_axis_name="c", subcore_axis_name="s")


@functools.partial(
    pl.kernel,
    mesh=_mesh,
    out_type=jax.ShapeDtypeStruct((SEQ, KB, NW, 8, BLK), jnp.float32),
    scratch_types=[
        pltpu.VMEM((SEQ, BLK), jnp.int32),
        pltpu.VMEM((NBUF, BLK, EMB), jnp.float32),
        pltpu.VMEM((NBUF, EMB, PITCH), jnp.float32),
    ]
    + [pltpu.SemaphoreType.DMA] * (2 * NBUF),
    compiler_params=pltpu.CompilerParams(
        use_tc_tiling_on_sc=False, needs_layout_passes=False),
)
def _emb_lookup(tok_hbm, table_hbm, out_hbm, idx_v, gbuf, sbuf, *sems):
    gsem = sems[:NBUF]
    ssem = sems[NBUF:]
    wid = lax.axis_index("s") * NC + lax.axis_index("c")
    pltpu.sync_copy(tok_hbm.at[:, wid], idx_v)
    iota16 = lax.iota(jnp.int32, 16)
    rows_q = [iota16 + q * 16 for q in range(EMB // 16)]

    def start_gather(s, b):
        pltpu.async_copy(table_hbm.at[idx_v.at[s]], gbuf.at[b], gsem[b])

    def wait_gather(s, b):
        pltpu.make_async_copy(table_hbm.at[idx_v.at[s]], gbuf.at[b],
                              gsem[b]).wait()

    def start_store(s, b):
        for kb in range(KB):
            pltpu.async_copy(sbuf.at[b, pl.ds(kb * 8, 8), pl.ds(0, BLK)],
                             out_hbm.at[s, kb, wid], ssem[b])

    def wait_store(s, b):
        for kb in range(KB):
            pltpu.make_async_copy(sbuf.at[b, pl.ds(kb * 8, 8), pl.ds(0, BLK)],
                                  out_hbm.at[s, kb, wid], ssem[b]).wait()

    def xform(b):
        @plsc.parallel_loop(0, BLK, 1, unroll=8)
        def tok_body(tb):
            cols = iota16 * 0 + tb
            for q in range(EMB // 16):
                v = gbuf[b, tb, pl.ds(q * 16, 16)]
                plsc.store_scatter(sbuf.at[b], [rows_q[q], cols], v)

    for b in range(NBUF):
        start_gather(b, b)
    for b in range(NBUF):
        wait_gather(b, b)
        xform(b)
        start_gather(b + NBUF, b)
        start_store(b, b)

    def body(t, carry):
        for b in range(NBUF):
            s = t * NBUF + b
            wait_gather(s, b)
            wait_store(s, b)
            xform(b)

            @pl.when(s + NBUF < SEQ)
            def _():
                start_gather(s + NBUF, b)

            start_store(s, b)
        return carry

    lax.fori_loop(1, SEQ // NBUF, body, 0)

    for b in range(NBUF):
        wait_store(SEQ - NBUF + b, b)


def _tc_scale_transpose(lo_ref, hi_ref, out_ref):
    w = jnp.concatenate([lo_ref[...], hi_ref[...]], axis=0)
    out_ref[...] = jnp.transpose(w) * SCALE


_scale_transpose = pl.pallas_call(
    _tc_scale_transpose,
    grid=(NTBLK,),
    in_specs=[
        pl.BlockSpec((EMB, TCHUNK), lambda i: (0, i)),
        pl.BlockSpec((EMB, TCHUNK),
                     lambda i: (0, jnp.minimum(
                         i + NTBLK, (VOCAB_ROWS + TCHUNK - 1) // TCHUNK - 1))),
    ],
    out_specs=pl.BlockSpec((TCHUNK, 2 * EMB), lambda i: (i, 0)),
    out_shape=jax.ShapeDtypeStruct((PHALF, 2 * EMB), jnp.float32),
)


def kernel(tokens, table):
    tok = tokens.astype(jnp.int32)
    tok = jnp.where(tok < PHALF, 2 * tok, 2 * tok - (STORED_ROWS - 1))
    tok_t = tok.T.reshape(SEQ, NW, BLK)
    tt = table.T
    scaled = _scale_transpose(tt, tt).reshape(STORED_ROWS, EMB)
    flat = _emb_lookup(tok_t, scaled)
    return flat.transpose(2, 4, 0, 1, 3).reshape(B_TOK, SEQ, EMB)
